# Initial kernel scaffold; baseline (speedup 1.0000x reference)
import jax
import jax.numpy as jnp
from jax.experimental import pallas as pl


def kernel(x, W_p, b_p):
    raise NotImplementedError("write your pallas kernel here")


# speedup vs baseline: 1.7042x; 1.7042x over previous
import functools

import jax
import jax.numpy as jnp
from jax import lax
from jax.experimental import pallas as pl
from jax.experimental.pallas import tpu as pltpu
from jax.experimental.pallas import tpu_sc as plsc

D = H = W = 96
PH = PW = 98
PLANE = PH * PW
PLANE_PAD = PLANE + 256
HW = H * W
NVOX = D * HW
CHVOL = PH * PH * PW


def _tc_coord_body(p0_ref, p1_ref, p2_ref, w_ref, b_ref, pyx_ref,
                   out_ref, cols_ref):
    planes = (p0_ref, p1_ref, p2_ref)
    for dz in range(3):
        for dy in range(3):
            for dx in range(3):
                tap = (dz * 3 + dy) * 3 + dx
                s = dy * PW + dx
                cols_ref[pl.ds(tap * 4, 4), :] = planes[dz][0, :, pl.ds(s, PLANE)]
    conv = jax.lax.dot_general(
        w_ref[...], cols_ref[...],
        dimension_numbers=(((1,), (0,)), ((), ())),
        precision=jax.lax.Precision.DEFAULT,
        preferred_element_type=jnp.float32,
    )
    off = conv + b_ref[...]
    zf = (pl.program_id(0) + 1).astype(jnp.float32)
    for y in range(96):
        seg = off[:, y * PW:y * PW + 96]
        out_ref[0, 0:3, y * 96:(y + 1) * 96] = seg[0:3, :] + zf
        out_ref[0, 3:6, y * 96:(y + 1) * 96] = (
            seg[3:6, :] + pyx_ref[0:1, y * 96:(y + 1) * 96])
        out_ref[0, 6:9, y * 96:(y + 1) * 96] = (
            seg[6:9, :] + pyx_ref[1:2, y * 96:(y + 1) * 96])


def _tc_coords(xpad_zmaj, w_tap, b_p, pyx):
    def plane_spec(dz):
        return pl.BlockSpec((1, 4, PLANE_PAD), lambda z, dz=dz: (z + dz, 0, 0))
    return pl.pallas_call(
        _tc_coord_body,
        grid=(96,),
        in_specs=[
            plane_spec(0), plane_spec(1), plane_spec(2),
            pl.BlockSpec((9, 108), lambda z: (0, 0)),
            pl.BlockSpec((9, 1), lambda z: (0, 0)),
            pl.BlockSpec((2, HW), lambda z: (0, 0)),
        ],
        out_specs=pl.BlockSpec((1, 9, HW), lambda z: (z, 0, 0)),
        out_shape=jax.ShapeDtypeStruct((96, 9, HW), jnp.float32),
        scratch_shapes=[pltpu.VMEM((108, PLANE), jnp.float32)],
    )(xpad_zmaj, xpad_zmaj, xpad_zmaj, w_tap, b_p, pyx)


NC, NS, L = 2, 16, 16
NW = NC * NS
CB = 1536
CPP = HW // CB
NCHUNK = D * CPP // NW

_C1 = float(PLANE)
_C2 = float(PW)


def _sc_interp_body(p_hbm, xf_hbm, out_hbm,
                    pzb, pyb, pxb,
                    idx0, idx1, idx2, idx3, idx4, idx5, idx6, idx7,
                    val0, val1, val2, val3, val4, val5, val6, val7,
                    wbuf, obuf, sem):
    cid = lax.axis_index("c")
    sid = lax.axis_index("s")
    wid = sid * NC + cid
    idxs = (idx0, idx1, idx2, idx3, idx4, idx5, idx6, idx7)
    vals = (val0, val1, val2, val3, val4, val5, val6, val7)

    for ch in range(3):
        chbase = ch * CHVOL

        def chunk_body(k, _, ch=ch, chbase=chbase):
            m = wid * NCHUNK + k
            z = m // CPP
            off = (m % CPP) * CB
            zrow = z * 9 * HW + off
            pltpu.sync_copy(p_hbm.at[pl.ds(zrow + ch * HW, CB)], pzb)
            pltpu.sync_copy(p_hbm.at[pl.ds(zrow + (3 + ch) * HW, CB)], pyb)
            pltpu.sync_copy(p_hbm.at[pl.ds(zrow + (6 + ch) * HW, CB)], pxb)

            def vec_body(i, _):
                sl = pl.ds(i * L, L)
                pz = pzb[sl]
                py = pyb[sl]
                px = pxb[sl]
                qz = jnp.minimum(jnp.maximum(pz, 0.0), 96.0)
                qy = jnp.minimum(jnp.maximum(py, 0.0), 96.0)
                qx = jnp.minimum(jnp.maximum(px, 0.0), 96.0)
                fz = qz.astype(jnp.int32).astype(jnp.float32) - qz
                fy = qy.astype(jnp.int32).astype(jnp.float32) - qy
                fx = qx.astype(jnp.int32).astype(jnp.float32) - qx
                az = 1.0 + fz
                bz = -fz
                ay = 1.0 + fy
                by = -fy
                ax = 1.0 + fx
                bx = -fx
                tz0 = qz * _C1
                tz1 = (qz + 1.0) * _C1
                ty0 = qy * _C2
                ty1 = (qy + 1.0) * _C2
                tx0 = qx
                tx1 = qx + 1.0
                corners = (
                    (tz0, ty0, tx0, az, ay, ax),
                    (tz1, ty0, tx0, bz, ay, ax),
                    (tz0, ty1, tx0, az, by, ax),
                    (tz0, ty0, tx1, az, ay, bx),
                    (tz1, ty1, tx0, bz, by, ax),
                    (tz1, ty0, tx1, bz, ay, bx),
                    (tz0, ty1, tx1, az, by, bx),
                    (tz1, ty1, tx1, bz, by, bx),
                )
                for j, (tz, ty, tx, wz, wy, wx) in enumerate(corners):
                    idx = ((tz + ty) + tx).astype(jnp.int32) + chbase
                    idxs[j][sl] = idx
                    wbuf[j, sl] = (wz * wy) * wx
                return 0

            lax.fori_loop(0, CB // L, vec_body, 0, unroll=2)

            cps = [pltpu.async_copy(xf_hbm.at[idxs[j]], vals[j], sem)
                   for j in range(8)]
            for cp in cps:
                cp.wait()

            def mix_body(i, _):
                sl = pl.ds(i * L, L)
                acc = wbuf[0, sl] * vals[0][sl]
                for j in range(1, 8):
                    acc = acc + wbuf[j, sl] * vals[j][sl]
                obuf[sl] = acc
                return 0

            lax.fori_loop(0, CB // L, mix_body, 0, unroll=2)
            pltpu.sync_copy(obuf, out_hbm.at[pl.ds(ch * NVOX + z * HW + off, CB)])
            return 0

        lax.fori_loop(0, NCHUNK, chunk_body, 0)


def _sc_interp(P, xf):
    mesh = plsc.VectorSubcoreMesh(core_axis_name="c", subcore_axis_name="s")
    kern = pl.kernel(
        _sc_interp_body,
        mesh=mesh,
        out_type=jax.ShapeDtypeStruct((3 * NVOX,), jnp.float32),
        scratch_types=(
            [pltpu.VMEM((CB,), jnp.float32)] * 3
            + [pltpu.VMEM((CB,), jnp.int32)] * 8
            + [pltpu.VMEM((CB,), jnp.float32)] * 8
            + [pltpu.VMEM((8, CB), jnp.float32),
               pltpu.VMEM((CB,), jnp.float32),
               pltpu.SemaphoreType.DMA]
        ),
    )
    return kern(P, xf)


def kernel(x, W_p, b_p):
    x3 = x[0]
    xpad = jnp.pad(x3, ((0, 0), (1, 1), (1, 1), (1, 1)))
    xpad_zmaj = jnp.pad(
        jnp.transpose(xpad.reshape(4, PH, PLANE), (1, 0, 2)),
        ((0, 0), (0, 0), (0, 256)))
    w_tap = jnp.transpose(W_p.reshape(9, 4, 27), (0, 2, 1)).reshape(9, 108)
    yy = jnp.repeat(jnp.arange(1, 97, dtype=jnp.float32), 96)
    xx = jnp.tile(jnp.arange(1, 97, dtype=jnp.float32), 96)
    pyx = jnp.stack([yy, xx], axis=0)
    P = _tc_coords(xpad_zmaj, w_tap, b_p.reshape(9, 1), pyx)
    xf = xpad[:3].reshape(3 * CHVOL)
    out3 = _sc_interp(P.reshape(-1), xf)
    return jnp.concatenate([out3.reshape(1, 3, D, H, W), x[:, 3:4]], axis=1)


# speedup vs baseline: 1.9130x; 1.1225x over previous
import functools

import jax
import jax.numpy as jnp
from jax import lax
from jax.experimental import pallas as pl
from jax.experimental.pallas import tpu as pltpu
from jax.experimental.pallas import tpu_sc as plsc

D = H = W = 96
PH = PW = 98
PLANE = PH * PW
PLANE_PAD = PLANE + 256
HW = H * W
NVOX = D * HW
CHVOL = PH * PH * PW


def _tc_coord_body(p0_ref, p1_ref, p2_ref, w_ref, b_ref, pyx_ref,
                   out_ref, cols_ref):
    planes = (p0_ref, p1_ref, p2_ref)
    for dz in range(3):
        for dy in range(3):
            for dx in range(3):
                tap = (dz * 3 + dy) * 3 + dx
                s = dy * PW + dx
                cols_ref[pl.ds(tap * 4, 4), :] = planes[dz][0, :, pl.ds(s, PLANE)]
    conv = jax.lax.dot_general(
        w_ref[...], cols_ref[...],
        dimension_numbers=(((1,), (0,)), ((), ())),
        precision=jax.lax.Precision.DEFAULT,
        preferred_element_type=jnp.float32,
    )
    off = conv + b_ref[...]
    zf = (pl.program_id(0) + 1).astype(jnp.float32)
    for y in range(96):
        seg = off[:, y * PW:y * PW + 96]
        out_ref[0, 0:3, y * 96:(y + 1) * 96] = seg[0:3, :] + zf
        out_ref[0, 3:6, y * 96:(y + 1) * 96] = (
            seg[3:6, :] + pyx_ref[0:1, y * 96:(y + 1) * 96])
        out_ref[0, 6:9, y * 96:(y + 1) * 96] = (
            seg[6:9, :] + pyx_ref[1:2, y * 96:(y + 1) * 96])


def _tc_coords(xpad_zmaj, w_tap, b_p, pyx):
    def plane_spec(dz):
        return pl.BlockSpec((1, 4, PLANE_PAD), lambda z, dz=dz: (z + dz, 0, 0))
    return pl.pallas_call(
        _tc_coord_body,
        grid=(96,),
        in_specs=[
            plane_spec(0), plane_spec(1), plane_spec(2),
            pl.BlockSpec((9, 108), lambda z: (0, 0)),
            pl.BlockSpec((9, 1), lambda z: (0, 0)),
            pl.BlockSpec((2, HW), lambda z: (0, 0)),
        ],
        out_specs=pl.BlockSpec((1, 9, HW), lambda z: (z, 0, 0)),
        out_shape=jax.ShapeDtypeStruct((96, 9, HW), jnp.float32),
        scratch_shapes=[pltpu.VMEM((108, PLANE), jnp.float32)],
    )(xpad_zmaj, xpad_zmaj, xpad_zmaj, w_tap, b_p, pyx)


NC, NS, L = 2, 16, 16
NW = NC * NS
CB = 1536
CPP = HW // CB
NCHUNK = D * CPP // NW

_C1 = float(PLANE)
_C2 = float(PW)


NG = 3 * D * CPP // NW


def _sc_interp_body(p_hbm, xf_hbm, out_hbm,
                    pzb, pyb, pxb,
                    idxA0, idxA1, idxA2, idxA3, idxA4, idxA5, idxA6, idxA7,
                    idxB0, idxB1, idxB2, idxB3, idxB4, idxB5, idxB6, idxB7,
                    valA0, valA1, valA2, valA3, valA4, valA5, valA6, valA7,
                    valB0, valB1, valB2, valB3, valB4, valB5, valB6, valB7,
                    wbufA, wbufB, obuf, semA, semB):
    cid = lax.axis_index("c")
    sid = lax.axis_index("s")
    wid = sid * NC + cid
    A = ((idxA0, idxA1, idxA2, idxA3, idxA4, idxA5, idxA6, idxA7),
         (valA0, valA1, valA2, valA3, valA4, valA5, valA6, valA7),
         wbufA, semA)
    B = ((idxB0, idxB1, idxB2, idxB3, idxB4, idxB5, idxB6, idxB7),
         (valB0, valB1, valB2, valB3, valB4, valB5, valB6, valB7),
         wbufB, semB)

    def locate(g):
        ch = g // NCHUNK
        k = g % NCHUNK
        m = wid * NCHUNK + k
        z = m // CPP
        off = (m % CPP) * CB
        return ch, z * 9 * HW + off, ch * NVOX + z * HW + off

    def compute_fire(g, bufs):
        idxs, vals, wbuf, sem = bufs
        ch, zrow, _ = locate(g)
        pltpu.sync_copy(p_hbm.at[pl.ds(zrow + ch * HW, CB)], pzb)
        pltpu.sync_copy(p_hbm.at[pl.ds(zrow + (3 + ch) * HW, CB)], pyb)
        pltpu.sync_copy(p_hbm.at[pl.ds(zrow + (6 + ch) * HW, CB)], pxb)
        chbase = ch * CHVOL

        def vec_body(i, _):
            sl = pl.ds(i * L, L)
            pz = pzb[sl]
            py = pyb[sl]
            px = pxb[sl]
            qz = jnp.minimum(jnp.maximum(pz, 0.0), 96.0)
            qy = jnp.minimum(jnp.maximum(py, 0.0), 96.0)
            qx = jnp.minimum(jnp.maximum(px, 0.0), 96.0)
            fz = qz.astype(jnp.int32).astype(jnp.float32) - qz
            fy = qy.astype(jnp.int32).astype(jnp.float32) - qy
            fx = qx.astype(jnp.int32).astype(jnp.float32) - qx
            az = 1.0 + fz
            bz = -fz
            ay = 1.0 + fy
            by = -fy
            ax = 1.0 + fx
            bx = -fx
            tz0 = qz * _C1
            tz1 = (qz + 1.0) * _C1
            ty0 = qy * _C2
            ty1 = (qy + 1.0) * _C2
            tx0 = qx
            tx1 = qx + 1.0
            corners = (
                (tz0, ty0, tx0, az, ay, ax),
                (tz1, ty0, tx0, bz, ay, ax),
                (tz0, ty1, tx0, az, by, ax),
                (tz0, ty0, tx1, az, ay, bx),
                (tz1, ty1, tx0, bz, by, ax),
                (tz1, ty0, tx1, bz, ay, bx),
                (tz0, ty1, tx1, az, by, bx),
                (tz1, ty1, tx1, bz, by, bx),
            )
            cb = jnp.int32(chbase)
            for j, (tz, ty, tx, wz, wy, wx) in enumerate(corners):
                idxs[j][sl] = ((tz + ty) + tx).astype(jnp.int32) + cb
                wbuf[j, sl] = (wz * wy) * wx
            return 0

        lax.fori_loop(0, CB // L, vec_body, 0, unroll=2)
        for j in range(8):
            pltpu.async_copy(xf_hbm.at[idxs[j]], vals[j], sem)

    def drain_mix_store(g, bufs):
        _, vals, wbuf, sem = bufs
        _, _, obase = locate(g)
        for j in range(8):
            pltpu.make_async_copy(p_hbm.at[pl.ds(0, CB)], vals[j], sem).wait()

        def mix_body(i, _):
            sl = pl.ds(i * L, L)
            acc = wbuf[0, sl] * vals[0][sl]
            for j in range(1, 8):
                acc = acc + wbuf[j, sl] * vals[j][sl]
            obuf[sl] = acc
            return 0

        lax.fori_loop(0, CB // L, mix_body, 0, unroll=2)
        pltpu.sync_copy(obuf, out_hbm.at[pl.ds(obase, CB)])

    compute_fire(jnp.int32(0), A)

    def pair_body(gp, _):
        g0 = gp * 2
        compute_fire(g0 + 1, B)
        drain_mix_store(g0, A)

        @pl.when(gp < NG // 2 - 1)
        def _():
            compute_fire(g0 + 2, A)
        drain_mix_store(g0 + 1, B)
        return 0

    lax.fori_loop(0, NG // 2, pair_body, 0)


def _sc_interp(P, xf):
    mesh = plsc.VectorSubcoreMesh(core_axis_name="c", subcore_axis_name="s")
    kern = pl.kernel(
        _sc_interp_body,
        mesh=mesh,
        out_type=jax.ShapeDtypeStruct((3 * NVOX,), jnp.float32),
        scratch_types=(
            [pltpu.VMEM((CB,), jnp.float32)] * 3
            + [pltpu.VMEM((CB,), jnp.int32)] * 16
            + [pltpu.VMEM((CB,), jnp.float32)] * 16
            + [pltpu.VMEM((8, CB), jnp.float32)] * 2
            + [pltpu.VMEM((CB,), jnp.float32),
               pltpu.SemaphoreType.DMA,
               pltpu.SemaphoreType.DMA]
        ),
    )
    return kern(P, xf)


def kernel(x, W_p, b_p):
    x3 = x[0]
    xpad = jnp.pad(x3, ((0, 0), (1, 1), (1, 1), (1, 1)))
    xpad_zmaj = jnp.pad(
        jnp.transpose(xpad.reshape(4, PH, PLANE), (1, 0, 2)),
        ((0, 0), (0, 0), (0, 256)))
    w_tap = jnp.transpose(W_p.reshape(9, 4, 27), (0, 2, 1)).reshape(9, 108)
    yy = jnp.repeat(jnp.arange(1, 97, dtype=jnp.float32), 96)
    xx = jnp.tile(jnp.arange(1, 97, dtype=jnp.float32), 96)
    pyx = jnp.stack([yy, xx], axis=0)
    P = _tc_coords(xpad_zmaj, w_tap, b_p.reshape(9, 1), pyx)
    xf = xpad[:3].reshape(3 * CHVOL)
    out3 = _sc_interp(P.reshape(-1), xf)
    return jnp.concatenate([out3.reshape(1, 3, D, H, W), x[:, 3:4]], axis=1)


# speedup vs baseline: 2.3198x; 1.2127x over previous
import functools

import jax
import jax.numpy as jnp
from jax import lax
from jax.experimental import pallas as pl
from jax.experimental.pallas import tpu as pltpu
from jax.experimental.pallas import tpu_sc as plsc

D = H = W = 96
PH = PW = 98
PLANE = PH * PW
PLANE_PAD = PLANE + 256
HW = H * W
NVOX = D * HW
CHVOL = PH * PH * PW
CHVOL_PAD = CHVOL + 120


def _tc_coord_body(p0_ref, p1_ref, p2_ref, w_ref, b_ref, pyx_ref,
                   out_ref, cols_ref):
    planes = (p0_ref, p1_ref, p2_ref)
    for dz in range(3):
        for dy in range(3):
            for dx in range(3):
                tap = (dz * 3 + dy) * 3 + dx
                s = dy * PW + dx
                cols_ref[pl.ds(tap * 4, 4), :] = planes[dz][0, :, pl.ds(s, PLANE)]
    conv = jax.lax.dot_general(
        w_ref[...], cols_ref[...],
        dimension_numbers=(((1,), (0,)), ((), ())),
        precision=jax.lax.Precision.DEFAULT,
        preferred_element_type=jnp.float32,
    )
    off = conv + b_ref[...]
    zf = (pl.program_id(0) + 1).astype(jnp.float32)
    for y in range(96):
        seg = off[:, y * PW:y * PW + 96]
        out_ref[0, 0:3, y * 96:(y + 1) * 96] = seg[0:3, :] + zf
        out_ref[0, 3:6, y * 96:(y + 1) * 96] = (
            seg[3:6, :] + pyx_ref[0:1, y * 96:(y + 1) * 96])
        out_ref[0, 6:9, y * 96:(y + 1) * 96] = (
            seg[6:9, :] + pyx_ref[1:2, y * 96:(y + 1) * 96])


def _tc_coords(xpad_zmaj, w_tap, b_p, pyx):
    def plane_spec(dz):
        return pl.BlockSpec((1, 4, PLANE_PAD), lambda z, dz=dz: (z + dz, 0, 0))
    return pl.pallas_call(
        _tc_coord_body,
        grid=(96,),
        in_specs=[
            plane_spec(0), plane_spec(1), plane_spec(2),
            pl.BlockSpec((9, 108), lambda z: (0, 0)),
            pl.BlockSpec((9, 1), lambda z: (0, 0)),
            pl.BlockSpec((2, HW), lambda z: (0, 0)),
        ],
        out_specs=pl.BlockSpec((1, 9, HW), lambda z: (z, 0, 0)),
        out_shape=jax.ShapeDtypeStruct((96, 9, HW), jnp.float32),
        scratch_shapes=[pltpu.VMEM((108, PLANE), jnp.float32)],
    )(xpad_zmaj, xpad_zmaj, xpad_zmaj, w_tap, b_p, pyx)


NC, NS, L = 2, 16, 16
NW = NC * NS
CB = 1152
CPP = HW // CB
NCHUNK = D * CPP // NW

_C1 = float(PLANE)
_C2 = float(PW)


NG = 3 * D * CPP // NW


def _sc_interp_body(p_hbm, xf_hbm, out_hbm,
                    pzb, pyb, pxb,
                    idxA0, idxA1, idxA2, idxA3, idxA4, idxA5, idxA6, idxA7,
                    idxB0, idxB1, idxB2, idxB3, idxB4, idxB5, idxB6, idxB7,
                    valA0, valA1, valA2, valA3, valA4, valA5, valA6, valA7,
                    valB0, valB1, valB2, valB3, valB4, valB5, valB6, valB7,
                    wbufA, wbufB, obuf, table, semA, semB):
    cid = lax.axis_index("c")
    sid = lax.axis_index("s")
    wid = sid * NC + cid
    A = ((idxA0, idxA1, idxA2, idxA3, idxA4, idxA5, idxA6, idxA7),
         (valA0, valA1, valA2, valA3, valA4, valA5, valA6, valA7),
         wbufA, semA)
    B = ((idxB0, idxB1, idxB2, idxB3, idxB4, idxB5, idxB6, idxB7),
         (valB0, valB1, valB2, valB3, valB4, valB5, valB6, valB7),
         wbufB, semB)

    def locate(g):
        ch = g // NCHUNK
        k = g % NCHUNK
        m = wid * NCHUNK + k
        z = m // CPP
        off = (m % CPP) * CB
        return ch, z * 9 * HW + off, ch * NVOX + z * HW + off

    def compute_fire(g, bufs):
        idxs, vals, wbuf, sem = bufs
        ch, zrow, _ = locate(g)
        pltpu.sync_copy(p_hbm.at[pl.ds(zrow + ch * HW, CB)], pzb)
        pltpu.sync_copy(p_hbm.at[pl.ds(zrow + (3 + ch) * HW, CB)], pyb)
        pltpu.sync_copy(p_hbm.at[pl.ds(zrow + (6 + ch) * HW, CB)], pxb)

        def vec_body(i, _):
            sl = pl.ds(i * L, L)
            pz = pzb[sl]
            py = pyb[sl]
            px = pxb[sl]
            qz = jnp.minimum(jnp.maximum(pz, 0.0), 96.0)
            qy = jnp.minimum(jnp.maximum(py, 0.0), 96.0)
            qx = jnp.minimum(jnp.maximum(px, 0.0), 96.0)
            fz = qz.astype(jnp.int32).astype(jnp.float32) - qz
            fy = qy.astype(jnp.int32).astype(jnp.float32) - qy
            fx = qx.astype(jnp.int32).astype(jnp.float32) - qx
            az = 1.0 + fz
            bz = -fz
            ay = 1.0 + fy
            by = -fy
            ax = 1.0 + fx
            bx = -fx
            tz0 = qz * _C1
            tz1 = (qz + 1.0) * _C1
            ty0 = qy * _C2
            ty1 = (qy + 1.0) * _C2
            tx0 = qx
            tx1 = qx + 1.0
            corners = (
                (tz0, ty0, tx0, az, ay, ax),
                (tz1, ty0, tx0, bz, ay, ax),
                (tz0, ty1, tx0, az, by, ax),
                (tz0, ty0, tx1, az, ay, bx),
                (tz1, ty1, tx0, bz, by, ax),
                (tz1, ty0, tx1, bz, ay, bx),
                (tz0, ty1, tx1, az, by, bx),
                (tz1, ty1, tx1, bz, by, bx),
            )
            for j, (tz, ty, tx, wz, wy, wx) in enumerate(corners):
                idxs[j][sl] = ((tz + ty) + tx).astype(jnp.int32)
                wbuf[j, sl] = (wz * wy) * wx
            return 0

        lax.fori_loop(0, CB // L, vec_body, 0, unroll=2)
        for j in range(8):
            pltpu.async_copy(table.at[idxs[j]], vals[j], sem)

    def drain_mix_store(g, bufs):
        _, vals, wbuf, sem = bufs
        _, _, obase = locate(g)
        for j in range(8):
            pltpu.make_async_copy(p_hbm.at[pl.ds(0, CB)], vals[j], sem).wait()

        def mix_body(i, _):
            sl = pl.ds(i * L, L)
            acc = wbuf[0, sl] * vals[0][sl]
            for j in range(1, 8):
                acc = acc + wbuf[j, sl] * vals[j][sl]
            obuf[sl] = acc
            return 0

        lax.fori_loop(0, CB // L, mix_body, 0, unroll=2)
        pltpu.sync_copy(obuf, out_hbm.at[pl.ds(obase, CB)])

    for ch in range(3):
        plsc.subcore_barrier()

        @pl.when(sid == 0)
        def _(ch=ch):
            pltpu.sync_copy(xf_hbm.at[pl.ds(ch * CHVOL_PAD, CHVOL_PAD)], table)
        plsc.subcore_barrier()

        gbase = jnp.int32(ch * NCHUNK)
        compute_fire(gbase, A)

        def pair_body(gp, _, gbase=gbase):
            g0 = gbase + gp * 2
            compute_fire(g0 + 1, B)
            drain_mix_store(g0, A)

            @pl.when(gp < NCHUNK // 2 - 1)
            def _():
                compute_fire(g0 + 2, A)
            drain_mix_store(g0 + 1, B)
            return 0

        lax.fori_loop(0, NCHUNK // 2, pair_body, 0)


def _sc_interp(P, xf):
    mesh = plsc.VectorSubcoreMesh(core_axis_name="c", subcore_axis_name="s")
    kern = pl.kernel(
        _sc_interp_body,
        mesh=mesh,
        out_type=jax.ShapeDtypeStruct((3 * NVOX,), jnp.float32),
        scratch_types=(
            [pltpu.VMEM((CB,), jnp.float32)] * 3
            + [pltpu.VMEM((CB,), jnp.int32)] * 16
            + [pltpu.VMEM((CB,), jnp.float32)] * 16
            + [pltpu.VMEM((8, CB), jnp.float32)] * 2
            + [pltpu.VMEM((CB,), jnp.float32),
               pltpu.VMEM_SHARED((CHVOL_PAD,), jnp.float32),
               pltpu.SemaphoreType.DMA,
               pltpu.SemaphoreType.DMA]
        ),
    )
    return kern(P, xf)


def kernel(x, W_p, b_p):
    x3 = x[0]
    xpad = jnp.pad(x3, ((0, 0), (1, 1), (1, 1), (1, 1)))
    xpad_zmaj = jnp.pad(
        jnp.transpose(xpad.reshape(4, PH, PLANE), (1, 0, 2)),
        ((0, 0), (0, 0), (0, 256)))
    w_tap = jnp.transpose(W_p.reshape(9, 4, 27), (0, 2, 1)).reshape(9, 108)
    yy = jnp.repeat(jnp.arange(1, 97, dtype=jnp.float32), 96)
    xx = jnp.tile(jnp.arange(1, 97, dtype=jnp.float32), 96)
    pyx = jnp.stack([yy, xx], axis=0)
    P = _tc_coords(xpad_zmaj, w_tap, b_p.reshape(9, 1), pyx)
    xf = jnp.pad(xpad[:3].reshape(3, CHVOL),
                 ((0, 0), (0, CHVOL_PAD - CHVOL))).reshape(3 * CHVOL_PAD)
    out3 = _sc_interp(P.reshape(-1), xf)
    return jnp.concatenate([out3.reshape(1, 3, D, H, W), x[:, 3:4]], axis=1)
